# baseline (speedup 1.0000x reference)
import jax
import jax.numpy as jnp
from jax.experimental import pallas as pl
from jax.experimental.pallas import tpu as pltpu

_IMG_H = 64
_IMG_W = 64
_NPIX = _IMG_H * _IMG_W
_N_CELLS = 64
_Z_WHAT = 32
_Z_WHERE = 4
_PER_CELL = 2 * _Z_WHAT + 2 * _Z_WHERE + 2 + 1
_SIGMA = 0.1
_PRIOR_PRES = 0.01
_SEQ_LEN = 10
_F = 4


_KTILE = 256
_NBUF = 8


def _enc_kernel(x_ref, w_hbm, b_ref, out_ref, wbuf, sems):
    k = pl.program_id(0)
    nk = pl.num_programs(0)

    def _start(blk, slot):
        pltpu.make_async_copy(
            w_hbm.at[pl.ds(blk * _KTILE, _KTILE), :],
            wbuf.at[slot], sems.at[slot]).start()

    @pl.when(k == 0)
    def _():
        for j in range(_NBUF):
            _start(j, j)

    slot = jax.lax.rem(k, _NBUF)
    pltpu.make_async_copy(
        w_hbm.at[pl.ds(k * _KTILE, _KTILE), :],
        wbuf.at[slot], sems.at[slot]).wait()
    acc = jnp.dot(x_ref[...], wbuf[slot],
                  preferred_element_type=jnp.float32)

    @pl.when(k == 0)
    def _():
        out_ref[...] = acc + b_ref[...]

    @pl.when(k > 0)
    def _():
        out_ref[...] += acc

    @pl.when(k + _NBUF < nk)
    def _():
        _start(k + _NBUF, slot)


def _kl_normal(mean, std):
    return 0.5 * (mean * mean + std * std - 2.0 * jnp.log(std) - 1.0)


def _dec_kernel(feat_ref, x_ref, aprev_ref, gw_ref, gb_ref, bgb_ref, pw_ref,
                eps_ref, y_ref, scal_ref):
    f = feat_ref[...]

    shift = jnp.dot(aprev_ref[:, 0, :], pw_ref[...],
                    preferred_element_type=jnp.float32)
    what_mean = f[:, :, 0:_Z_WHAT] + shift[:, None, :]
    what_std = jax.nn.softplus(f[:, :, _Z_WHAT:2 * _Z_WHAT]) + 1e-4
    o = 2 * _Z_WHAT
    where_mean = f[:, :, o:o + _Z_WHERE]
    where_std = jax.nn.softplus(f[:, :, o + _Z_WHERE:o + 2 * _Z_WHERE]) + 1e-4
    o2 = o + 2 * _Z_WHERE
    depth_mean = f[:, :, o2:o2 + 1]
    depth_std = jax.nn.softplus(f[:, :, o2 + 1:o2 + 2]) + 1e-4
    pres_logit = f[:, :, o2 + 2:o2 + 3]
    z_pres = jax.nn.sigmoid(pres_logit)
    w_depth = jax.nn.sigmoid(-depth_mean)

    g = jax.nn.sigmoid(
        jnp.dot(what_mean.reshape(_F * _N_CELLS, _Z_WHAT), gw_ref[...],
                preferred_element_type=jnp.float32)
        + gb_ref[...])

    alpha = g[:, 3 * _NPIX:4 * _NPIX].reshape(_F, _N_CELLS, _NPIX) * z_pres
    imp = alpha * w_depth
    imp_sum = jnp.sum(imp, axis=1, keepdims=True)
    alpha_sum = jnp.clip(jnp.sum(alpha, axis=1, keepdims=True), 0.0, 1.0)
    denom = imp_sum + 1e-5
    bg = jax.nn.sigmoid(bgb_ref[...])

    ys = []
    for ch in range(3):
        gch = g[:, ch * _NPIX:(ch + 1) * _NPIX].reshape(_F, _N_CELLS, _NPIX)
        num = jnp.sum(gch * imp, axis=1, keepdims=True)
        ys.append(num / denom * alpha_sum
                  + bg[ch:ch + 1, :][None] * (1.0 - alpha_sum))
    y = jnp.concatenate(ys, axis=1)
    y_ref[...] = y

    x = x_ref[...]
    diff = (x - y) / _SIGMA
    ll = (-0.5 * jnp.sum(diff * diff, axis=(1, 2), keepdims=True)
          + 3.0 * _NPIX * (-jnp.log(_SIGMA) - 0.5 * jnp.log(2.0 * jnp.pi)))

    def _red(v):
        return jnp.sum(v, axis=(1, 2), keepdims=True)

    kl_what = _red(_kl_normal(what_mean, what_std))
    kl_where = _red(_kl_normal(where_mean, where_std))
    kl_depth = _red(_kl_normal(depth_mean, depth_std))
    eps = eps_ref[0, 0]
    zp = jnp.clip(z_pres, eps, 1.0 - eps)
    kl_pres = _red(zp * jnp.log(zp / _PRIOR_PRES)
                   + (1.0 - zp) * jnp.log((1.0 - zp) / (1.0 - _PRIOR_PRES)))
    cnt = _red((z_pres > 0.7).astype(jnp.float32))
    zero = jnp.zeros((_F, 1, 1), jnp.float32)
    scal_ref[...] = jnp.concatenate(
        [ll, kl_what, kl_where, kl_depth, kl_pres, cnt, zero, zero], axis=2)


def kernel(seq, actions, enc_W, enc_b, glimpse_W, glimpse_b, bg_W, bg_b,
           prop_W, eps):
    bs, seq_len = seq.shape[0], seq.shape[1]
    m = bs * seq_len
    kdim = 3 * _NPIX
    ndim = _N_CELLS * _PER_CELL
    x_flat = seq.reshape(m, kdim)
    feat = pl.pallas_call(
        _enc_kernel,
        grid=(kdim // _KTILE,),
        in_specs=[
            pl.BlockSpec((m, _KTILE), lambda k: (0, k)),
            pl.BlockSpec(memory_space=pltpu.MemorySpace.HBM),
            pl.BlockSpec((1, ndim), lambda k: (0, 0)),
        ],
        out_specs=pl.BlockSpec((m, ndim), lambda k: (0, 0)),
        out_shape=jax.ShapeDtypeStruct((m, ndim), jnp.float32),
        scratch_shapes=[
            pltpu.VMEM((_NBUF, _KTILE, ndim), jnp.float32),
            pltpu.SemaphoreType.DMA((_NBUF,)),
        ],
    )(x_flat, enc_W, enc_b.reshape(1, ndim))

    feat_r = feat.reshape(m, _N_CELLS, _PER_CELL)
    x_r = seq.reshape(m, 3, _NPIX)
    aprev = jnp.concatenate(
        [jnp.zeros_like(actions[:, :1]), actions[:, :-1]], axis=1
    ).reshape(m, 1, actions.shape[-1])
    eps_arr = jnp.full((1, 8), eps, dtype=jnp.float32)

    y_flat, scal = pl.pallas_call(
        _dec_kernel,
        grid=(m // _F,),
        in_specs=[
            pl.BlockSpec((_F, _N_CELLS, _PER_CELL), lambda n: (n, 0, 0)),
            pl.BlockSpec((_F, 3, _NPIX), lambda n: (n, 0, 0)),
            pl.BlockSpec((_F, 1, 4), lambda n: (n, 0, 0)),
            pl.BlockSpec((_Z_WHAT, 4 * _NPIX), lambda n: (0, 0)),
            pl.BlockSpec((1, 4 * _NPIX), lambda n: (0, 0)),
            pl.BlockSpec((3, _NPIX), lambda n: (0, 0)),
            pl.BlockSpec((4, _Z_WHAT), lambda n: (0, 0)),
            pl.BlockSpec((1, 8), lambda n: (0, 0)),
        ],
        out_specs=[
            pl.BlockSpec((_F, 3, _NPIX), lambda n: (n, 0, 0)),
            pl.BlockSpec((_F, 1, 8), lambda n: (n, 0, 0)),
        ],
        out_shape=[
            jax.ShapeDtypeStruct((m, 3, _NPIX), jnp.float32),
            jax.ShapeDtypeStruct((m, 1, 8), jnp.float32),
        ],
    )(feat_r, x_r, aprev, glimpse_W, glimpse_b.reshape(1, 4 * _NPIX),
      bg_b.reshape(3, _NPIX), prop_W, eps_arr)

    y_seq = y_flat.reshape(bs, seq_len, 3, _IMG_H, _IMG_W)
    s = scal.reshape(bs, seq_len, 8)
    return (y_seq,
            s[..., 0].mean(axis=1),
            s[..., 1].mean(axis=1),
            s[..., 2].mean(axis=1),
            s[..., 3].mean(axis=1),
            s[..., 4].mean(axis=1),
            s[..., 5])


# speedup vs baseline: 1.0099x; 1.0099x over previous
import jax
import jax.numpy as jnp
from jax.experimental import pallas as pl
from jax.experimental.pallas import tpu as pltpu

_IMG_H = 64
_IMG_W = 64
_NPIX = _IMG_H * _IMG_W
_N_CELLS = 64
_Z_WHAT = 32
_Z_WHERE = 4
_PER_CELL = 2 * _Z_WHAT + 2 * _Z_WHERE + 2 + 1
_SIGMA = 0.1
_PRIOR_PRES = 0.01
_SEQ_LEN = 10
_F = 4


_KTILE = 256
_NBUF = 8


def _enc_kernel(x_ref, w_hbm, b_ref, out_ref, *scratch):
    wbufs, sems = scratch[:_NBUF], scratch[_NBUF]
    s = pl.program_id(0)
    ns = pl.num_programs(0)

    def _start(j, blk):
        pltpu.make_async_copy(
            w_hbm.at[pl.ds(blk * _KTILE, _KTILE), :],
            wbufs[j], sems.at[j]).start()

    @pl.when(s == 0)
    def _():
        for j in range(_NBUF):
            _start(j, j)

    acc = None
    for j in range(_NBUF):
        pltpu.make_async_copy(
            w_hbm.at[pl.ds((s * _NBUF + j) * _KTILE, _KTILE), :],
            wbufs[j], sems.at[j]).wait()
        part = jnp.dot(x_ref[:, j * _KTILE:(j + 1) * _KTILE], wbufs[j][...],
                       preferred_element_type=jnp.float32)
        acc = part if acc is None else acc + part

        @pl.when(s + 1 < ns)
        def _(j=j):
            _start(j, (s + 1) * _NBUF + j)

    @pl.when(s == 0)
    def _():
        out_ref[...] = acc + b_ref[...]

    @pl.when(s > 0)
    def _():
        out_ref[...] += acc


def _kl_normal(mean, std):
    return 0.5 * (mean * mean + std * std - 2.0 * jnp.log(std) - 1.0)


def _dec_kernel(feat_ref, x_ref, aprev_ref, gw_ref, gb_ref, bgb_ref, pw_ref,
                eps_ref, y_ref, scal_ref):
    f = feat_ref[...]

    shift = jnp.dot(aprev_ref[:, 0, :], pw_ref[...],
                    preferred_element_type=jnp.float32)
    what_mean = f[:, :, 0:_Z_WHAT] + shift[:, None, :]
    what_std = jax.nn.softplus(f[:, :, _Z_WHAT:2 * _Z_WHAT]) + 1e-4
    o = 2 * _Z_WHAT
    where_mean = f[:, :, o:o + _Z_WHERE]
    where_std = jax.nn.softplus(f[:, :, o + _Z_WHERE:o + 2 * _Z_WHERE]) + 1e-4
    o2 = o + 2 * _Z_WHERE
    depth_mean = f[:, :, o2:o2 + 1]
    depth_std = jax.nn.softplus(f[:, :, o2 + 1:o2 + 2]) + 1e-4
    pres_logit = f[:, :, o2 + 2:o2 + 3]
    z_pres = jax.nn.sigmoid(pres_logit)
    w_depth = jax.nn.sigmoid(-depth_mean)

    g = jax.nn.sigmoid(
        jnp.dot(what_mean.reshape(_F * _N_CELLS, _Z_WHAT), gw_ref[...],
                preferred_element_type=jnp.float32)
        + gb_ref[...])

    alpha = g[:, 3 * _NPIX:4 * _NPIX].reshape(_F, _N_CELLS, _NPIX) * z_pres
    imp = alpha * w_depth
    imp_sum = jnp.sum(imp, axis=1, keepdims=True)
    alpha_sum = jnp.clip(jnp.sum(alpha, axis=1, keepdims=True), 0.0, 1.0)
    denom = imp_sum + 1e-5
    bg = jax.nn.sigmoid(bgb_ref[...])

    ys = []
    for ch in range(3):
        gch = g[:, ch * _NPIX:(ch + 1) * _NPIX].reshape(_F, _N_CELLS, _NPIX)
        num = jnp.sum(gch * imp, axis=1, keepdims=True)
        ys.append(num / denom * alpha_sum
                  + bg[ch:ch + 1, :][None] * (1.0 - alpha_sum))
    y = jnp.concatenate(ys, axis=1)
    y_ref[...] = y

    x = x_ref[...]
    diff = (x - y) / _SIGMA
    ll = (-0.5 * jnp.sum(diff * diff, axis=(1, 2), keepdims=True)
          + 3.0 * _NPIX * (-jnp.log(_SIGMA) - 0.5 * jnp.log(2.0 * jnp.pi)))

    def _red(v):
        return jnp.sum(v, axis=(1, 2), keepdims=True)

    kl_what = _red(_kl_normal(what_mean, what_std))
    kl_where = _red(_kl_normal(where_mean, where_std))
    kl_depth = _red(_kl_normal(depth_mean, depth_std))
    eps = eps_ref[0, 0]
    zp = jnp.clip(z_pres, eps, 1.0 - eps)
    kl_pres = _red(zp * jnp.log(zp / _PRIOR_PRES)
                   + (1.0 - zp) * jnp.log((1.0 - zp) / (1.0 - _PRIOR_PRES)))
    cnt = _red((z_pres > 0.7).astype(jnp.float32))
    zero = jnp.zeros((_F, 1, 1), jnp.float32)
    scal_ref[...] = jnp.concatenate(
        [ll, kl_what, kl_where, kl_depth, kl_pres, cnt, zero, zero], axis=2)


def kernel(seq, actions, enc_W, enc_b, glimpse_W, glimpse_b, bg_W, bg_b,
           prop_W, eps):
    bs, seq_len = seq.shape[0], seq.shape[1]
    m = bs * seq_len
    kdim = 3 * _NPIX
    ndim = _N_CELLS * _PER_CELL
    x_flat = seq.reshape(m, kdim)
    feat = pl.pallas_call(
        _enc_kernel,
        grid=(kdim // (_KTILE * _NBUF),),
        in_specs=[
            pl.BlockSpec((m, _KTILE * _NBUF), lambda k: (0, k)),
            pl.BlockSpec(memory_space=pltpu.MemorySpace.HBM),
            pl.BlockSpec((1, ndim), lambda k: (0, 0)),
        ],
        out_specs=pl.BlockSpec((m, ndim), lambda k: (0, 0)),
        out_shape=jax.ShapeDtypeStruct((m, ndim), jnp.float32),
        scratch_shapes=(
            [pltpu.VMEM((_KTILE, ndim), jnp.float32) for _ in range(_NBUF)]
            + [pltpu.SemaphoreType.DMA((_NBUF,))]
        ),
    )(x_flat, enc_W, enc_b.reshape(1, ndim))

    feat_r = feat.reshape(m, _N_CELLS, _PER_CELL)
    x_r = seq.reshape(m, 3, _NPIX)
    aprev = jnp.concatenate(
        [jnp.zeros_like(actions[:, :1]), actions[:, :-1]], axis=1
    ).reshape(m, 1, actions.shape[-1])
    eps_arr = jnp.full((1, 8), eps, dtype=jnp.float32)

    y_flat, scal = pl.pallas_call(
        _dec_kernel,
        grid=(m // _F,),
        in_specs=[
            pl.BlockSpec((_F, _N_CELLS, _PER_CELL), lambda n: (n, 0, 0)),
            pl.BlockSpec((_F, 3, _NPIX), lambda n: (n, 0, 0)),
            pl.BlockSpec((_F, 1, 4), lambda n: (n, 0, 0)),
            pl.BlockSpec((_Z_WHAT, 4 * _NPIX), lambda n: (0, 0)),
            pl.BlockSpec((1, 4 * _NPIX), lambda n: (0, 0)),
            pl.BlockSpec((3, _NPIX), lambda n: (0, 0)),
            pl.BlockSpec((4, _Z_WHAT), lambda n: (0, 0)),
            pl.BlockSpec((1, 8), lambda n: (0, 0)),
        ],
        out_specs=[
            pl.BlockSpec((_F, 3, _NPIX), lambda n: (n, 0, 0)),
            pl.BlockSpec((_F, 1, 8), lambda n: (n, 0, 0)),
        ],
        out_shape=[
            jax.ShapeDtypeStruct((m, 3, _NPIX), jnp.float32),
            jax.ShapeDtypeStruct((m, 1, 8), jnp.float32),
        ],
    )(feat_r, x_r, aprev, glimpse_W, glimpse_b.reshape(1, 4 * _NPIX),
      bg_b.reshape(3, _NPIX), prop_W, eps_arr)

    y_seq = y_flat.reshape(bs, seq_len, 3, _IMG_H, _IMG_W)
    s = scal.reshape(bs, seq_len, 8)
    return (y_seq,
            s[..., 0].mean(axis=1),
            s[..., 1].mean(axis=1),
            s[..., 2].mean(axis=1),
            s[..., 3].mean(axis=1),
            s[..., 4].mean(axis=1),
            s[..., 5])


# speedup vs baseline: 1.0170x; 1.0070x over previous
import jax
import jax.numpy as jnp
from jax.experimental import pallas as pl
from jax.experimental.pallas import tpu as pltpu

_IMG_H = 64
_IMG_W = 64
_NPIX = _IMG_H * _IMG_W
_N_CELLS = 64
_Z_WHAT = 32
_Z_WHERE = 4
_PER_CELL = 2 * _Z_WHAT + 2 * _Z_WHERE + 2 + 1
_SIGMA = 0.1
_PRIOR_PRES = 0.01
_SEQ_LEN = 10
_F = 4


_KTILE = 256
_NBUF = 8


def _enc_kernel(x_ref, w_hbm, b_ref, out_ref, *scratch):
    wbufs, sems = scratch[:_NBUF], scratch[_NBUF]
    s = pl.program_id(0)
    ns = pl.num_programs(0)

    def _start(j, blk):
        pltpu.async_copy(
            w_hbm.at[pl.ds(blk * _KTILE, _KTILE), :],
            wbufs[j], sems.at[j], priority=j % 2)

    @pl.when(s == 0)
    def _():
        for j in range(_NBUF):
            _start(j, j)

    acc = None
    for j in range(_NBUF):
        pltpu.make_async_copy(
            w_hbm.at[pl.ds((s * _NBUF + j) * _KTILE, _KTILE), :],
            wbufs[j], sems.at[j]).wait()
        part = jnp.dot(x_ref[:, j * _KTILE:(j + 1) * _KTILE], wbufs[j][...],
                       preferred_element_type=jnp.float32)
        acc = part if acc is None else acc + part

        @pl.when(s + 1 < ns)
        def _(j=j):
            _start(j, (s + 1) * _NBUF + j)

    @pl.when(s == 0)
    def _():
        out_ref[...] = acc + b_ref[...]

    @pl.when(s > 0)
    def _():
        out_ref[...] += acc


def _kl_normal(mean, std):
    return 0.5 * (mean * mean + std * std - 2.0 * jnp.log(std) - 1.0)


def _dec_kernel(feat_ref, x_ref, aprev_ref, gw_ref, gb_ref, bgb_ref, pw_ref,
                eps_ref, y_ref, scal_ref):
    f = feat_ref[...]

    shift = jnp.dot(aprev_ref[:, 0, :], pw_ref[...],
                    preferred_element_type=jnp.float32)
    what_mean = f[:, :, 0:_Z_WHAT] + shift[:, None, :]
    what_std = jax.nn.softplus(f[:, :, _Z_WHAT:2 * _Z_WHAT]) + 1e-4
    o = 2 * _Z_WHAT
    where_mean = f[:, :, o:o + _Z_WHERE]
    where_std = jax.nn.softplus(f[:, :, o + _Z_WHERE:o + 2 * _Z_WHERE]) + 1e-4
    o2 = o + 2 * _Z_WHERE
    depth_mean = f[:, :, o2:o2 + 1]
    depth_std = jax.nn.softplus(f[:, :, o2 + 1:o2 + 2]) + 1e-4
    pres_logit = f[:, :, o2 + 2:o2 + 3]
    z_pres = jax.nn.sigmoid(pres_logit)
    w_depth = jax.nn.sigmoid(-depth_mean)

    g = jax.nn.sigmoid(
        jnp.dot(what_mean.reshape(_F * _N_CELLS, _Z_WHAT), gw_ref[...],
                preferred_element_type=jnp.float32)
        + gb_ref[...])

    alpha = g[:, 3 * _NPIX:4 * _NPIX].reshape(_F, _N_CELLS, _NPIX) * z_pres
    imp = alpha * w_depth
    imp_sum = jnp.sum(imp, axis=1, keepdims=True)
    alpha_sum = jnp.clip(jnp.sum(alpha, axis=1, keepdims=True), 0.0, 1.0)
    denom = imp_sum + 1e-5
    bg = jax.nn.sigmoid(bgb_ref[...])

    ys = []
    for ch in range(3):
        gch = g[:, ch * _NPIX:(ch + 1) * _NPIX].reshape(_F, _N_CELLS, _NPIX)
        num = jnp.sum(gch * imp, axis=1, keepdims=True)
        ys.append(num / denom * alpha_sum
                  + bg[ch:ch + 1, :][None] * (1.0 - alpha_sum))
    y = jnp.concatenate(ys, axis=1)
    y_ref[...] = y

    x = x_ref[...]
    diff = (x - y) / _SIGMA
    ll = (-0.5 * jnp.sum(diff * diff, axis=(1, 2), keepdims=True)
          + 3.0 * _NPIX * (-jnp.log(_SIGMA) - 0.5 * jnp.log(2.0 * jnp.pi)))

    def _red(v):
        return jnp.sum(v, axis=(1, 2), keepdims=True)

    kl_what = _red(_kl_normal(what_mean, what_std))
    kl_where = _red(_kl_normal(where_mean, where_std))
    kl_depth = _red(_kl_normal(depth_mean, depth_std))
    eps = eps_ref[0, 0]
    zp = jnp.clip(z_pres, eps, 1.0 - eps)
    kl_pres = _red(zp * jnp.log(zp / _PRIOR_PRES)
                   + (1.0 - zp) * jnp.log((1.0 - zp) / (1.0 - _PRIOR_PRES)))
    cnt = _red((z_pres > 0.7).astype(jnp.float32))
    zero = jnp.zeros((_F, 1, 1), jnp.float32)
    scal_ref[...] = jnp.concatenate(
        [ll, kl_what, kl_where, kl_depth, kl_pres, cnt, zero, zero], axis=2)


def kernel(seq, actions, enc_W, enc_b, glimpse_W, glimpse_b, bg_W, bg_b,
           prop_W, eps):
    bs, seq_len = seq.shape[0], seq.shape[1]
    m = bs * seq_len
    kdim = 3 * _NPIX
    ndim = _N_CELLS * _PER_CELL
    x_flat = seq.reshape(m, kdim)
    feat = pl.pallas_call(
        _enc_kernel,
        grid=(kdim // (_KTILE * _NBUF),),
        in_specs=[
            pl.BlockSpec((m, _KTILE * _NBUF), lambda k: (0, k)),
            pl.BlockSpec(memory_space=pltpu.MemorySpace.HBM),
            pl.BlockSpec((1, ndim), lambda k: (0, 0)),
        ],
        out_specs=pl.BlockSpec((m, ndim), lambda k: (0, 0)),
        out_shape=jax.ShapeDtypeStruct((m, ndim), jnp.float32),
        scratch_shapes=(
            [pltpu.VMEM((_KTILE, ndim), jnp.float32) for _ in range(_NBUF)]
            + [pltpu.SemaphoreType.DMA((_NBUF,))]
        ),
    )(x_flat, enc_W, enc_b.reshape(1, ndim))

    feat_r = feat.reshape(m, _N_CELLS, _PER_CELL)
    x_r = seq.reshape(m, 3, _NPIX)
    aprev = jnp.concatenate(
        [jnp.zeros_like(actions[:, :1]), actions[:, :-1]], axis=1
    ).reshape(m, 1, actions.shape[-1])
    eps_arr = jnp.full((1, 8), eps, dtype=jnp.float32)

    y_flat, scal = pl.pallas_call(
        _dec_kernel,
        grid=(m // _F,),
        in_specs=[
            pl.BlockSpec((_F, _N_CELLS, _PER_CELL), lambda n: (n, 0, 0)),
            pl.BlockSpec((_F, 3, _NPIX), lambda n: (n, 0, 0)),
            pl.BlockSpec((_F, 1, 4), lambda n: (n, 0, 0)),
            pl.BlockSpec((_Z_WHAT, 4 * _NPIX), lambda n: (0, 0)),
            pl.BlockSpec((1, 4 * _NPIX), lambda n: (0, 0)),
            pl.BlockSpec((3, _NPIX), lambda n: (0, 0)),
            pl.BlockSpec((4, _Z_WHAT), lambda n: (0, 0)),
            pl.BlockSpec((1, 8), lambda n: (0, 0)),
        ],
        out_specs=[
            pl.BlockSpec((_F, 3, _NPIX), lambda n: (n, 0, 0)),
            pl.BlockSpec((_F, 1, 8), lambda n: (n, 0, 0)),
        ],
        out_shape=[
            jax.ShapeDtypeStruct((m, 3, _NPIX), jnp.float32),
            jax.ShapeDtypeStruct((m, 1, 8), jnp.float32),
        ],
    )(feat_r, x_r, aprev, glimpse_W, glimpse_b.reshape(1, 4 * _NPIX),
      bg_b.reshape(3, _NPIX), prop_W, eps_arr)

    y_seq = y_flat.reshape(bs, seq_len, 3, _IMG_H, _IMG_W)
    s = scal.reshape(bs, seq_len, 8)
    return (y_seq,
            s[..., 0].mean(axis=1),
            s[..., 1].mean(axis=1),
            s[..., 2].mean(axis=1),
            s[..., 3].mean(axis=1),
            s[..., 4].mean(axis=1),
            s[..., 5])
